# baseline (speedup 1.0000x reference)
import jax
import jax.numpy as jnp
from jax.experimental import pallas as pl

N = 200
H = 7
C = 300
K = 20
HI = jax.lax.Precision.HIGHEST
NEG = -1e30


def _fused(x_ref, wggl_ref, bggl_ref, ein_ref, eout_ref, wgat_ref,
           asad_ref, bgat_ref, gamma_ref, beta_ref, w3_ref, b3_ref,
           out_ref):
    f32 = jnp.float32
    x = x_ref[...]

    z = jnp.dot(x, wggl_ref[...], precision=HI) + bggl_ref[...]
    atrr = 1.0 / (1.0 + jnp.exp(-z))
    A = jax.lax.dot_general(atrr, atrr, (((1,), (1,)), ((), ())), precision=HI)

    row_i = jax.lax.broadcasted_iota(jnp.int32, (N, N), 0)
    col_j = jax.lax.broadcasted_iota(jnp.int32, (N, N), 1)

    a_work = A
    for _ in range(K):
        colmax = jnp.max(a_work, axis=0, keepdims=True)
        cand = jnp.where(a_work == colmax, row_i, N)
        jstar = jnp.min(cand, axis=0, keepdims=True)
        a_work = jnp.where(row_i == jstar, NEG, a_work)
    adjt = jnp.where(a_work == NEG, 1.0, 0.0)

    ones_col = jnp.ones((N, 1), f32)
    in_deg = jnp.dot(adjt, ones_col, precision=HI)
    in_idx = jnp.minimum(in_deg, float(N - 1))
    onehot_in = (col_j.astype(f32) == in_idx).astype(f32)
    in_f = jnp.dot(onehot_in, ein_ref[...], precision=HI)
    onehot_out = (col_j[:1, :] == K).astype(f32)
    orow = jnp.dot(onehot_out, eout_ref[...], precision=HI)
    out_f = jnp.broadcast_to(orow, (N, 8))

    in_cat = jnp.concatenate([x, in_f, out_f], axis=1)
    h = jnp.dot(in_cat.astype(jnp.bfloat16), wgat_ref[...],
                preferred_element_type=f32)
    t2 = jax.lax.dot_general(asad_ref[...], h, (((0,), (1,)), ((), ())))
    es = jnp.transpose(t2[:H, :])
    ed_t = t2[H:, :]

    eye_f = (row_i == col_j).astype(f32)
    dist0 = 2.0 * eye_f - 1.0

    iota_col = jax.lax.broadcasted_iota(jnp.int32, (N, 1), 0)

    def bfs_cond(carry):
        return carry[3] != 0

    def _hop(d, dist, frontier):
        allowed = frontier * jnp.where(d < iota_col, 1.0, 0.0)
        reach = jax.lax.dot_general(allowed, adjt, (((1,), (1,)), ((), ())))
        nxt = jnp.where((reach > 0.0) & (dist == -1.0), 1.0, 0.0)
        dist = jnp.where(nxt > 0.0, (d + 1).astype(f32), dist)
        return dist, nxt

    def bfs_body(carry):
        d, dist, frontier, _ = carry
        dist, nxt = _hop(d, dist, frontier)
        dist, nxt = _hop(d + 1, dist, nxt)
        go = jnp.where(jnp.any(nxt > 0.0), jnp.int32(1), jnp.int32(0))
        return d + 2, dist, nxt, go

    dist, nxt = _hop(jnp.int32(0), dist0, eye_f)
    for dd in range(1, 4):
        dist, nxt = _hop(jnp.int32(dd), dist, nxt)
    go0 = jnp.where(jnp.any(nxt > 0.0), jnp.int32(1), jnp.int32(0))
    _, dist, _, _ = jax.lax.while_loop(
        bfs_cond, bfs_body,
        (jnp.int32(4), dist, nxt, go0))
    bias = jnp.where(dist != -1.0, dist, NEG)

    outs = []
    for hh in range(H):
        es_col = es[:, hh:hh + 1]
        ed_row = ed_t[hh:hh + 1, :]
        v = es_col + ed_row
        logit = jnp.maximum(v, 0.2 * v) + bias
        m = jnp.max(logit, axis=0, keepdims=True)
        e = jnp.exp(logit - m)
        den = jnp.sum(e, axis=0, keepdims=True)
        alpha = e * (1.0 / (den + 1e-16))
        hcol = h[:, hh * C:(hh + 1) * C]
        outs.append(jax.lax.dot_general(
            alpha, hcol, (((0,), (0,)), ((), ()))))
    out = jnp.concatenate(outs, axis=1) + bgat_ref[...]

    mu = jnp.mean(out, axis=0, keepdims=True)
    ctr = out - mu
    var = jnp.mean(ctr * ctr, axis=0, keepdims=True)
    out = ctr * jax.lax.rsqrt(var + 1e-5) * gamma_ref[...] + beta_ref[...]

    res = jnp.dot(out.astype(jnp.bfloat16), w3_ref[...],
                  preferred_element_type=f32) + b3_ref[...]
    out_ref[...] = jnp.maximum(res, 0.0)


def kernel(x, W_ggl, b_ggl, emb_in, emb_out, W_gat, a_src, a_dst, b_gat,
           gamma, beta, W3, b3):
    eyeH = jnp.eye(H, dtype=jnp.float32)
    as_mat = (a_src[:, :, None] * eyeH[:, None, :]).reshape(H * C, H)
    ad_mat = (a_dst[:, :, None] * eyeH[:, None, :]).reshape(H * C, H)
    asad_mat = jnp.concatenate([as_mat, ad_mat], axis=1)
    return pl.pallas_call(
        _fused,
        out_shape=jax.ShapeDtypeStruct((N, 256), jnp.float32),
    )(x, W_ggl, b_ggl.reshape(1, -1), emb_in, emb_out,
      W_gat.astype(jnp.bfloat16), asad_mat, b_gat.reshape(1, -1),
      gamma.reshape(1, -1), beta.reshape(1, -1),
      W3.astype(jnp.bfloat16), b3.reshape(1, -1))


# speedup vs baseline: 1.1630x; 1.1630x over previous
import jax
import jax.numpy as jnp
from jax.experimental import pallas as pl
from jax.experimental.pallas import tpu as pltpu

N = 200
H = 7
C = 300
K = 20
HI = jax.lax.Precision.HIGHEST
NEG = -1e30


def _fused(x_ref, wggl_ref, bggl_ref, ein_ref, eout_ref, wgat_hbm,
           asad_ref, bgat_ref, gamma_ref, beta_ref, w3_hbm, b3_ref,
           out_ref, wgat_vmem, w3_vmem, sem1, sem2):
    f32 = jnp.float32
    x = x_ref[...]

    cp1 = pltpu.make_async_copy(wgat_hbm, wgat_vmem, sem1)
    cp2 = pltpu.make_async_copy(w3_hbm, w3_vmem, sem2)
    cp1.start()
    cp2.start()

    z = jnp.dot(x, wggl_ref[...], precision=HI) + bggl_ref[...]
    atrr = 1.0 / (1.0 + jnp.exp(-z))
    A = jax.lax.dot_general(atrr, atrr, (((1,), (1,)), ((), ())), precision=HI)

    row_i = jax.lax.broadcasted_iota(jnp.int32, (N, N), 0)
    col_j = jax.lax.broadcasted_iota(jnp.int32, (N, N), 1)

    a_work = A
    for _ in range(K):
        colmax = jnp.max(a_work, axis=0, keepdims=True)
        cand = jnp.where(a_work == colmax, row_i, N)
        jstar = jnp.min(cand, axis=0, keepdims=True)
        a_work = jnp.where(row_i == jstar, NEG, a_work)
    adjt = jnp.where(a_work == NEG, 1.0, 0.0)

    ones_col = jnp.ones((N, 1), f32)
    in_deg = jnp.dot(adjt, ones_col, precision=HI)
    in_idx = jnp.minimum(in_deg, float(N - 1))
    onehot_in = (col_j.astype(f32) == in_idx).astype(f32)
    in_f = jnp.dot(onehot_in, ein_ref[...], precision=HI)
    onehot_out = (col_j[:1, :] == K).astype(f32)
    orow = jnp.dot(onehot_out, eout_ref[...], precision=HI)
    out_f = jnp.broadcast_to(orow, (N, 8))

    in_cat = jnp.concatenate([x, in_f, out_f], axis=1)
    cp1.wait()
    h = jnp.dot(in_cat, wgat_vmem[...])
    t2 = jax.lax.dot_general(asad_ref[...], h, (((0,), (1,)), ((), ())))
    es = jnp.transpose(t2[:H, :])
    ed_t = t2[H:, :]

    eye_f = (row_i == col_j).astype(f32)
    dist0 = 2.0 * eye_f - 1.0

    iota_col = jax.lax.broadcasted_iota(jnp.int32, (N, 1), 0)

    def bfs_cond(carry):
        return carry[3] != 0

    def _hop(d, dist, frontier):
        allowed = frontier * jnp.where(d < iota_col, 1.0, 0.0)
        reach = jax.lax.dot_general(allowed, adjt, (((1,), (1,)), ((), ())))
        nxt = jnp.where((reach > 0.0) & (dist == -1.0), 1.0, 0.0)
        dist = jnp.where(nxt > 0.0, (d + 1).astype(f32), dist)
        return dist, nxt

    def bfs_body(carry):
        d, dist, frontier, _ = carry
        dist, nxt = _hop(d, dist, frontier)
        dist, nxt = _hop(d + 1, dist, nxt)
        go = jnp.where(jnp.any(nxt > 0.0), jnp.int32(1), jnp.int32(0))
        return d + 2, dist, nxt, go

    dist, nxt = _hop(jnp.int32(0), dist0, eye_f)
    for dd in range(1, 4):
        dist, nxt = _hop(jnp.int32(dd), dist, nxt)
    go0 = jnp.where(jnp.any(nxt > 0.0), jnp.int32(1), jnp.int32(0))
    _, dist, _, _ = jax.lax.while_loop(
        bfs_cond, bfs_body,
        (jnp.int32(4), dist, nxt, go0))
    bias = jnp.where(dist != -1.0, dist, NEG)

    outs = []
    for hh in range(H):
        es_col = es[:, hh:hh + 1]
        ed_row = ed_t[hh:hh + 1, :]
        v = es_col + ed_row
        logit = jnp.maximum(v, 0.2 * v) + bias
        m = jnp.max(logit, axis=0, keepdims=True)
        e = jnp.exp(logit - m)
        den = jnp.sum(e, axis=0, keepdims=True)
        alpha = e * (1.0 / (den + 1e-16))
        hcol = h[:, hh * C:(hh + 1) * C]
        outs.append(jax.lax.dot_general(
            alpha, hcol, (((0,), (0,)), ((), ()))))
    out = jnp.concatenate(outs, axis=1) + bgat_ref[...]

    mu = jnp.mean(out, axis=0, keepdims=True)
    ctr = out - mu
    var = jnp.mean(ctr * ctr, axis=0, keepdims=True)
    out = ctr * jax.lax.rsqrt(var + 1e-5) * gamma_ref[...] + beta_ref[...]

    cp2.wait()
    res = jnp.dot(out, w3_vmem[...]) + b3_ref[...]
    out_ref[...] = jnp.maximum(res, 0.0)


def kernel(x, W_ggl, b_ggl, emb_in, emb_out, W_gat, a_src, a_dst, b_gat,
           gamma, beta, W3, b3):
    eyeH = jnp.eye(H, dtype=jnp.float32)
    as_mat = (a_src[:, :, None] * eyeH[:, None, :]).reshape(H * C, H)
    ad_mat = (a_dst[:, :, None] * eyeH[:, None, :]).reshape(H * C, H)
    asad_mat = jnp.concatenate([as_mat, ad_mat], axis=1)
    nargs = 12
    specs = [pl.BlockSpec(memory_space=pl.ANY)
             if i in (5, 10) else pl.BlockSpec(memory_space=pltpu.MemorySpace.VMEM)
             for i in range(nargs)]
    return pl.pallas_call(
        _fused,
        out_shape=jax.ShapeDtypeStruct((N, 256), jnp.float32),
        in_specs=specs,
        scratch_shapes=[pltpu.VMEM((272, H * C), jnp.float32),
                        pltpu.VMEM((H * C, 256), jnp.float32),
                        pltpu.SemaphoreType.DMA,
                        pltpu.SemaphoreType.DMA],
    )(x, W_ggl, b_ggl.reshape(1, -1), emb_in, emb_out, W_gat,
      asad_mat, b_gat.reshape(1, -1), gamma.reshape(1, -1),
      beta.reshape(1, -1), W3, b3.reshape(1, -1))
